# baseline (speedup 1.0000x reference)
import functools

import jax
import jax.numpy as jnp
import numpy as np
from jax import lax
from jax.experimental import pallas as pl
from jax.experimental.pallas import tpu as pltpu
from jax.experimental.pallas import tpu_sc as plsc

_A = 9
_FEAT_STRIDE = 16
_PRE_N = 6000
_POST_N = 300
_THRESH = 0.7
_KA = 36864
_B = 4
_NPAD = 6144
_ROWS = 8
_COLS = _NPAD // _ROWS
_SEL_COLS = _KA // _ROWS
_INT_MIN = np.int32(-2**31)
_DUMP = _B * _NPAD
_GROWS = _B * _NPAD + 8
_TPT = _KA // 8
_NBATCH = _TPT // 128


def _np_generate_anchors():
    base_size = 16
    ratios = np.array([0.5, 1.0, 2.0])
    scales = np.array([8.0, 16.0, 32.0])

    def whctrs(a):
        w = a[2] - a[0] + 1.0
        h = a[3] - a[1] + 1.0
        return w, h, a[0] + 0.5 * (w - 1.0), a[1] + 0.5 * (h - 1.0)

    def mkanchors(ws, hs, xc, yc):
        ws = ws[:, None]
        hs = hs[:, None]
        return np.hstack((xc - 0.5 * (ws - 1.0), yc - 0.5 * (hs - 1.0),
                          xc + 0.5 * (ws - 1.0), yc + 0.5 * (hs - 1.0)))

    base = np.array([1.0, 1.0, base_size, base_size], dtype=np.float64) - 1.0
    w, h, xc, yc = whctrs(base)
    size = w * h
    ws = np.round(np.sqrt(size / ratios))
    hs = np.round(ws * ratios)
    ra = mkanchors(ws, hs, xc, yc)
    outs = []
    for i in range(ra.shape[0]):
        w, h, xc, yc = whctrs(ra[i])
        outs.append(mkanchors(w * scales, h * scales, xc, yc))
    return np.vstack(outs)


def _posn_body(scf_ref, out_ref):
    b = pl.program_id(0)
    s = scf_ref[0, :, :]
    bits = lax.bitcast_convert_type(s, jnp.int32)
    key = jnp.where(bits >= 0, bits, bits ^ jnp.int32(0x7FFFFFFF))
    ri = lax.broadcasted_iota(jnp.int32, (_ROWS, _SEL_COLS), 0)
    ci = lax.broadcasted_iota(jnp.int32, (_ROWS, _SEL_COLS), 1)
    fi = ri * _SEL_COLS + ci

    pu = jnp.int32(0)
    for j in range(31, -1, -1):
        bit = jnp.int32(np.int32(np.uint32(1) << np.uint32(j)))
        qu = pu | bit
        qs = qu ^ _INT_MIN
        cnt = jnp.sum((key >= qs).astype(jnp.int32))
        pu = jnp.where(cnt >= _PRE_N, qu, pu)
    t_s = pu ^ _INT_MIN

    cgt = jnp.sum((key > t_s).astype(jnp.int32))
    need = _PRE_N - cgt
    tie = key == t_s
    pi = jnp.int32(0)
    for j in range(15, -1, -1):
        qi = pi | jnp.int32(1 << j)
        cnt = jnp.sum((tie & (fi <= qi)).astype(jnp.int32))
        pi = jnp.where(cnt <= need, qi, pi)

    member = (key > t_s) | (tie & (fi <= pi))
    mi = member.astype(jnp.int32)
    c = mi
    sh = 1
    while sh < _SEL_COLS:
        z = jnp.zeros((_ROWS, sh), jnp.int32)
        c = c + jnp.concatenate([z, c[:, :_SEL_COLS - sh]], axis=1)
        sh *= 2
    rowtot = c[:, _SEL_COLS - 1:_SEL_COLS]
    acc = jnp.zeros((1, 1), jnp.int32)
    offs = []
    for r in range(_ROWS):
        offs.append(acc)
        acc = acc + rowtot[r:r + 1, :]
    ro = jnp.concatenate(offs, axis=0)
    excl = c - mi + ro
    out_ref[0, :, :] = jnp.where(member, b * _NPAD + excl,
                                 jnp.int32(_DUMP))


def _posn_pallas(scf):
    return pl.pallas_call(
        _posn_body,
        grid=(_B,),
        in_specs=[pl.BlockSpec((1, _ROWS, _SEL_COLS), lambda i: (i, 0, 0))],
        out_specs=pl.BlockSpec((1, _ROWS, _SEL_COLS), lambda i: (i, 0, 0)),
        out_shape=jax.ShapeDtypeStruct((_B, _ROWS, _SEL_COLS), jnp.int32),
    )(scf)


def _sc_scatter(scf_hbm, fpos_hbm, props_hbm, gbox_hbm, gsc_hbm,
                sbuf, pbuf, fposb, sem):
    nc = 2
    wid = lax.axis_index("s") * nc + lax.axis_index("c")
    pltpu.sync_copy(scf_hbm.at[pl.ds(wid * _TPT, _TPT)], sbuf)
    pltpu.sync_copy(props_hbm.at[pl.ds(wid * _TPT, _TPT)], pbuf)
    pltpu.sync_copy(fpos_hbm.at[pl.ds(wid * _NBATCH, _NBATCH)], fposb)

    def body(j, _):
        pltpu.async_copy(pbuf.at[pl.ds(j * 128, 128)],
                         gbox_hbm.at[fposb.at[j]], sem).wait()
        pltpu.async_copy(sbuf.at[pl.ds(j * 128, 128)],
                         gsc_hbm.at[fposb.at[j]], sem).wait()
        return 0

    lax.fori_loop(0, _NBATCH, body, 0)


def _sc_pallas(scf_flat, fpos2d, props_flat):
    mesh = plsc.VectorSubcoreMesh(core_axis_name="c", subcore_axis_name="s")
    f = pl.kernel(
        _sc_scatter,
        mesh=mesh,
        out_type=[
            jax.ShapeDtypeStruct((_GROWS, 4), jnp.float32),
            jax.ShapeDtypeStruct((_GROWS,), jnp.float32),
        ],
        scratch_types=[
            pltpu.VMEM((_TPT,), jnp.float32),
            pltpu.VMEM((_TPT, 4), jnp.float32),
            pltpu.VMEM((_NBATCH, 128), jnp.int32),
            pltpu.SemaphoreType.DMA,
        ],
        compiler_params=pltpu.CompilerParams(use_tc_tiling_on_sc=False),
    )
    return f(scf_flat, fpos2d, props_flat)


def _nms_body(boxes_ref, scores_ref, out_ref, area_ref):
    ri = lax.broadcasted_iota(jnp.int32, (_ROWS, _COLS), 0)
    ci = lax.broadcasted_iota(jnp.int32, (_ROWS, _COLS), 1)
    fi = ri * _COLS + ci
    neg_inf = jnp.float32(-jnp.inf)
    zero = jnp.float32(0.0)
    big = jnp.int32(2**30)

    s0s = []
    for b in range(_B):
        x1 = boxes_ref[b, 0, :, :]
        y1 = boxes_ref[b, 1, :, :]
        x2 = boxes_ref[b, 2, :, :]
        y2 = boxes_ref[b, 3, :, :]
        area_ref[b, :, :] = (x2 - x1 + 1.0) * (y2 - y1 + 1.0)
        s0s.append(jnp.where(fi < _PRE_N, scores_ref[b, :, :], neg_inf))

    def step(t, ss):
        ms = [jnp.max(ss[b]) for b in range(_B)]
        eqs = [ss[b] == ms[b] for b in range(_B)]
        idxs = [jnp.min(jnp.where(eqs[b], fi, big)) for b in range(_B)]
        sels = [fi == idxs[b] for b in range(_B)]
        x1s = [boxes_ref[b, 0, :, :] for b in range(_B)]
        y1s = [boxes_ref[b, 1, :, :] for b in range(_B)]
        x2s = [boxes_ref[b, 2, :, :] for b in range(_B)]
        y2s = [boxes_ref[b, 3, :, :] for b in range(_B)]
        bx1s = [jnp.sum(jnp.where(sels[b], x1s[b], zero)) for b in range(_B)]
        by1s = [jnp.sum(jnp.where(sels[b], y1s[b], zero)) for b in range(_B)]
        bx2s = [jnp.sum(jnp.where(sels[b], x2s[b], zero)) for b in range(_B)]
        by2s = [jnp.sum(jnp.where(sels[b], y2s[b], zero)) for b in range(_B)]
        out = []
        for b in range(_B):
            x1, y1, x2, y2 = x1s[b], y1s[b], x2s[b], y2s[b]
            areas = area_ref[b, :, :]
            bx1, by1, bx2, by2 = bx1s[b], by1s[b], bx2s[b], by2s[b]
            barea = (bx2 - bx1 + 1.0) * (by2 - by1 + 1.0)
            xx1 = jnp.maximum(x1, bx1)
            yy1 = jnp.maximum(y1, by1)
            xx2 = jnp.minimum(x2, bx2)
            yy2 = jnp.minimum(y2, by2)
            inter = (jnp.maximum(0.0, xx2 - xx1 + 1.0)
                     * jnp.maximum(0.0, yy2 - yy1 + 1.0))
            iou = inter / jnp.maximum(areas + barea - inter, 1e-6)
            out.append(jnp.where(iou <= _THRESH, ss[b], neg_inf))
        for b in range(_B):
            okf = jnp.isfinite(ms[b]).astype(jnp.float32)
            out_ref[b, t, 0] = bx1s[b] * okf
            out_ref[b, t, 1] = by1s[b] * okf
            out_ref[b, t, 2] = bx2s[b] * okf
            out_ref[b, t, 3] = by2s[b] * okf
        return tuple(out)

    lax.fori_loop(0, _POST_N, step, tuple(s0s))


def _nms_pallas(boxes, scores):
    B = boxes.shape[0]
    return pl.pallas_call(
        _nms_body,
        out_specs=pl.BlockSpec(memory_space=pltpu.SMEM),
        out_shape=jax.ShapeDtypeStruct((B, _POST_N, 4), jnp.float32),
        scratch_shapes=[pltpu.VMEM((_B, _ROWS, _COLS), jnp.float32)],
    )(boxes, scores)


def _bbox_transform_inv(boxes, deltas):
    widths = boxes[..., 2] - boxes[..., 0] + 1.0
    heights = boxes[..., 3] - boxes[..., 1] + 1.0
    ctr_x = boxes[..., 0] + 0.5 * widths
    ctr_y = boxes[..., 1] + 0.5 * heights
    dx, dy, dw, dh = (deltas[..., 0], deltas[..., 1],
                      deltas[..., 2], deltas[..., 3])
    pcx = dx * widths + ctr_x
    pcy = dy * heights + ctr_y
    pw = jnp.exp(dw) * widths
    ph = jnp.exp(dh) * heights
    return jnp.stack([pcx - 0.5 * pw, pcy - 0.5 * ph,
                      pcx + 0.5 * pw, pcy + 0.5 * ph], axis=-1)


def _clip_boxes(boxes, im_info):
    hmax = (im_info[:, 0] - 1.0)[:, None]
    wmax = (im_info[:, 1] - 1.0)[:, None]
    x1 = jnp.clip(boxes[..., 0], 0.0, wmax)
    y1 = jnp.clip(boxes[..., 1], 0.0, hmax)
    x2 = jnp.clip(boxes[..., 2], 0.0, wmax)
    y2 = jnp.clip(boxes[..., 3], 0.0, hmax)
    return jnp.stack([x1, y1, x2, y2], axis=-1)


def kernel(scores, bbox_deltas, im_info, cfg_key):
    del cfg_key
    anchors = jnp.asarray(_np_generate_anchors(), dtype=scores.dtype)
    sc = scores[:, _A:, :, :]
    B = bbox_deltas.shape[0]
    fh, fw = sc.shape[2], sc.shape[3]
    sx, sy = jnp.meshgrid(jnp.arange(fw) * _FEAT_STRIDE,
                          jnp.arange(fh) * _FEAT_STRIDE)
    shifts = jnp.stack([sx.ravel(), sy.ravel(), sx.ravel(), sy.ravel()],
                       axis=1).astype(scores.dtype)
    K = shifts.shape[0]
    all_anchors = (shifts[:, None, :] + anchors[None, :, :]).reshape(K * _A, 4)
    all_anchors = jnp.broadcast_to(all_anchors[None], (B, K * _A, 4))
    deltas = jnp.transpose(bbox_deltas, (0, 2, 3, 1)).reshape(B, -1, 4)
    scf = jnp.transpose(sc, (0, 2, 3, 1)).reshape(B, -1)
    proposals = _clip_boxes(_bbox_transform_inv(all_anchors, deltas), im_info)

    fpos = _posn_pallas(scf.reshape(B, _ROWS, _SEL_COLS))
    gbox, gsc = _sc_pallas(scf.reshape(-1),
                           fpos.reshape(_B * _ROWS * _NBATCH, 128),
                           proposals.reshape(B * _KA, 4))
    gbox = gbox[:_B * _NPAD].reshape(B, _NPAD, 4)
    gsc = gsc[:_B * _NPAD]
    boxes_in = jnp.transpose(gbox, (0, 2, 1)).reshape(B, 4, _ROWS, _COLS)
    scores_in = gsc.reshape(B, _ROWS, _COLS)

    kept = _nms_pallas(boxes_in, scores_in)
    bcol = jnp.broadcast_to(
        jnp.arange(B, dtype=kept.dtype)[:, None, None], (B, _POST_N, 1))
    return jnp.concatenate([bcol, kept], axis=2)


# speedup vs baseline: 1.0104x; 1.0104x over previous
import functools

import jax
import jax.numpy as jnp
import numpy as np
from jax import lax
from jax.experimental import pallas as pl
from jax.experimental.pallas import tpu as pltpu
from jax.experimental.pallas import tpu_sc as plsc

_A = 9
_FEAT_STRIDE = 16
_PRE_N = 6000
_POST_N = 300
_THRESH = 0.7
_KA = 36864
_B = 4
_NPAD = 6144
_ROWS = 8
_COLS = _NPAD // _ROWS
_SEL_COLS = _KA // _ROWS
_INT_MIN = np.int32(-2**31)
_DUMP = _B * _NPAD
_GROWS = _B * _NPAD + 128
_TPT = _KA // 8
_NBATCH = _TPT // 128


def _np_generate_anchors():
    base_size = 16
    ratios = np.array([0.5, 1.0, 2.0])
    scales = np.array([8.0, 16.0, 32.0])

    def whctrs(a):
        w = a[2] - a[0] + 1.0
        h = a[3] - a[1] + 1.0
        return w, h, a[0] + 0.5 * (w - 1.0), a[1] + 0.5 * (h - 1.0)

    def mkanchors(ws, hs, xc, yc):
        ws = ws[:, None]
        hs = hs[:, None]
        return np.hstack((xc - 0.5 * (ws - 1.0), yc - 0.5 * (hs - 1.0),
                          xc + 0.5 * (ws - 1.0), yc + 0.5 * (hs - 1.0)))

    base = np.array([1.0, 1.0, base_size, base_size], dtype=np.float64) - 1.0
    w, h, xc, yc = whctrs(base)
    size = w * h
    ws = np.round(np.sqrt(size / ratios))
    hs = np.round(ws * ratios)
    ra = mkanchors(ws, hs, xc, yc)
    outs = []
    for i in range(ra.shape[0]):
        w, h, xc, yc = whctrs(ra[i])
        outs.append(mkanchors(w * scales, h * scales, xc, yc))
    return np.vstack(outs)


def _posn_body(scf_ref, out_ref):
    b = pl.program_id(0)
    s = scf_ref[0, :, :]
    bits = lax.bitcast_convert_type(s, jnp.int32)
    key = jnp.where(bits >= 0, bits, bits ^ jnp.int32(0x7FFFFFFF))
    ri = lax.broadcasted_iota(jnp.int32, (_ROWS, _SEL_COLS), 0)
    ci = lax.broadcasted_iota(jnp.int32, (_ROWS, _SEL_COLS), 1)
    fi = ri * _SEL_COLS + ci

    pu = jnp.int32(0)
    for j in range(31, -1, -1):
        bit = jnp.int32(np.int32(np.uint32(1) << np.uint32(j)))
        qu = pu | bit
        qs = qu ^ _INT_MIN
        cnt = jnp.sum((key >= qs).astype(jnp.int32))
        pu = jnp.where(cnt >= _PRE_N, qu, pu)
    t_s = pu ^ _INT_MIN

    cgt = jnp.sum((key > t_s).astype(jnp.int32))
    need = _PRE_N - cgt
    tie = key == t_s
    pi = jnp.int32(0)
    for j in range(15, -1, -1):
        qi = pi | jnp.int32(1 << j)
        cnt = jnp.sum((tie & (fi <= qi)).astype(jnp.int32))
        pi = jnp.where(cnt <= need, qi, pi)

    member = (key > t_s) | (tie & (fi <= pi))
    mi = member.astype(jnp.int32)
    c = mi
    sh = 1
    while sh < _SEL_COLS:
        z = jnp.zeros((_ROWS, sh), jnp.int32)
        c = c + jnp.concatenate([z, c[:, :_SEL_COLS - sh]], axis=1)
        sh *= 2
    rowtot = c[:, _SEL_COLS - 1:_SEL_COLS]
    acc = jnp.zeros((1, 1), jnp.int32)
    offs = []
    for r in range(_ROWS):
        offs.append(acc)
        acc = acc + rowtot[r:r + 1, :]
    ro = jnp.concatenate(offs, axis=0)
    excl = c - mi + ro
    dump = _DUMP + (fi & jnp.int32(127))
    out_ref[0, :, :] = jnp.where(member, b * _NPAD + excl, dump)


def _posn_pallas(scf):
    return pl.pallas_call(
        _posn_body,
        grid=(_B,),
        in_specs=[pl.BlockSpec((1, _ROWS, _SEL_COLS), lambda i: (i, 0, 0))],
        out_specs=pl.BlockSpec((1, _ROWS, _SEL_COLS), lambda i: (i, 0, 0)),
        out_shape=jax.ShapeDtypeStruct((_B, _ROWS, _SEL_COLS), jnp.int32),
    )(scf)


def _sc_scatter(scf_hbm, fpos_hbm, props_hbm, gbox_hbm, gsc_hbm,
                sbuf, pbuf, fposb, sem):
    nc = 2
    wid = lax.axis_index("s") * nc + lax.axis_index("c")
    pltpu.sync_copy(scf_hbm.at[pl.ds(wid * _TPT, _TPT)], sbuf)
    pltpu.sync_copy(props_hbm.at[pl.ds(wid * _TPT, _TPT)], pbuf)
    pltpu.sync_copy(fpos_hbm.at[pl.ds(wid * _NBATCH, _NBATCH)], fposb)

    def body(j, _):
        cp1 = pltpu.async_copy(pbuf.at[pl.ds(j * 128, 128)],
                               gbox_hbm.at[fposb.at[j]], sem)
        cp2 = pltpu.async_copy(sbuf.at[pl.ds(j * 128, 128)],
                               gsc_hbm.at[fposb.at[j]], sem)
        cp1.wait()
        cp2.wait()
        return 0

    lax.fori_loop(0, _NBATCH, body, 0)


def _sc_pallas(scf_flat, fpos2d, props_flat):
    mesh = plsc.VectorSubcoreMesh(core_axis_name="c", subcore_axis_name="s")
    f = pl.kernel(
        _sc_scatter,
        mesh=mesh,
        out_type=[
            jax.ShapeDtypeStruct((_GROWS, 4), jnp.float32),
            jax.ShapeDtypeStruct((_GROWS,), jnp.float32),
        ],
        scratch_types=[
            pltpu.VMEM((_TPT,), jnp.float32),
            pltpu.VMEM((_TPT, 4), jnp.float32),
            pltpu.VMEM((_NBATCH, 128), jnp.int32),
            pltpu.SemaphoreType.DMA,
        ],
        compiler_params=pltpu.CompilerParams(use_tc_tiling_on_sc=False),
    )
    return f(scf_flat, fpos2d, props_flat)


def _nms_body(boxes_ref, scores_ref, out_ref, area_ref):
    ri = lax.broadcasted_iota(jnp.int32, (_ROWS, _COLS), 0)
    ci = lax.broadcasted_iota(jnp.int32, (_ROWS, _COLS), 1)
    fi = ri * _COLS + ci
    neg_inf = jnp.float32(-jnp.inf)
    zero = jnp.float32(0.0)
    big = jnp.int32(2**30)

    s0s = []
    for b in range(_B):
        x1 = boxes_ref[b, 0, :, :]
        y1 = boxes_ref[b, 1, :, :]
        x2 = boxes_ref[b, 2, :, :]
        y2 = boxes_ref[b, 3, :, :]
        area_ref[b, :, :] = (x2 - x1 + 1.0) * (y2 - y1 + 1.0)
        s0s.append(jnp.where(fi < _PRE_N, scores_ref[b, :, :], neg_inf))

    def step(t, ss):
        ms = [jnp.max(ss[b]) for b in range(_B)]
        eqs = [ss[b] == ms[b] for b in range(_B)]
        idxs = [jnp.min(jnp.where(eqs[b], fi, big)) for b in range(_B)]
        sels = [fi == idxs[b] for b in range(_B)]
        x1s = [boxes_ref[b, 0, :, :] for b in range(_B)]
        y1s = [boxes_ref[b, 1, :, :] for b in range(_B)]
        x2s = [boxes_ref[b, 2, :, :] for b in range(_B)]
        y2s = [boxes_ref[b, 3, :, :] for b in range(_B)]
        bx1s = [jnp.sum(jnp.where(sels[b], x1s[b], zero)) for b in range(_B)]
        by1s = [jnp.sum(jnp.where(sels[b], y1s[b], zero)) for b in range(_B)]
        bx2s = [jnp.sum(jnp.where(sels[b], x2s[b], zero)) for b in range(_B)]
        by2s = [jnp.sum(jnp.where(sels[b], y2s[b], zero)) for b in range(_B)]
        out = []
        for b in range(_B):
            x1, y1, x2, y2 = x1s[b], y1s[b], x2s[b], y2s[b]
            areas = area_ref[b, :, :]
            bx1, by1, bx2, by2 = bx1s[b], by1s[b], bx2s[b], by2s[b]
            barea = (bx2 - bx1 + 1.0) * (by2 - by1 + 1.0)
            xx1 = jnp.maximum(x1, bx1)
            yy1 = jnp.maximum(y1, by1)
            xx2 = jnp.minimum(x2, bx2)
            yy2 = jnp.minimum(y2, by2)
            inter = (jnp.maximum(0.0, xx2 - xx1 + 1.0)
                     * jnp.maximum(0.0, yy2 - yy1 + 1.0))
            iou = inter / jnp.maximum(areas + barea - inter, 1e-6)
            out.append(jnp.where(iou <= _THRESH, ss[b], neg_inf))
        for b in range(_B):
            okf = jnp.isfinite(ms[b]).astype(jnp.float32)
            out_ref[b, t, 0] = bx1s[b] * okf
            out_ref[b, t, 1] = by1s[b] * okf
            out_ref[b, t, 2] = bx2s[b] * okf
            out_ref[b, t, 3] = by2s[b] * okf
        return tuple(out)

    lax.fori_loop(0, _POST_N, step, tuple(s0s))


def _nms_pallas(boxes, scores):
    B = boxes.shape[0]
    return pl.pallas_call(
        _nms_body,
        out_specs=pl.BlockSpec(memory_space=pltpu.SMEM),
        out_shape=jax.ShapeDtypeStruct((B, _POST_N, 4), jnp.float32),
        scratch_shapes=[pltpu.VMEM((_B, _ROWS, _COLS), jnp.float32)],
    )(boxes, scores)


def _bbox_transform_inv(boxes, deltas):
    widths = boxes[..., 2] - boxes[..., 0] + 1.0
    heights = boxes[..., 3] - boxes[..., 1] + 1.0
    ctr_x = boxes[..., 0] + 0.5 * widths
    ctr_y = boxes[..., 1] + 0.5 * heights
    dx, dy, dw, dh = (deltas[..., 0], deltas[..., 1],
                      deltas[..., 2], deltas[..., 3])
    pcx = dx * widths + ctr_x
    pcy = dy * heights + ctr_y
    pw = jnp.exp(dw) * widths
    ph = jnp.exp(dh) * heights
    return jnp.stack([pcx - 0.5 * pw, pcy - 0.5 * ph,
                      pcx + 0.5 * pw, pcy + 0.5 * ph], axis=-1)


def _clip_boxes(boxes, im_info):
    hmax = (im_info[:, 0] - 1.0)[:, None]
    wmax = (im_info[:, 1] - 1.0)[:, None]
    x1 = jnp.clip(boxes[..., 0], 0.0, wmax)
    y1 = jnp.clip(boxes[..., 1], 0.0, hmax)
    x2 = jnp.clip(boxes[..., 2], 0.0, wmax)
    y2 = jnp.clip(boxes[..., 3], 0.0, hmax)
    return jnp.stack([x1, y1, x2, y2], axis=-1)


def kernel(scores, bbox_deltas, im_info, cfg_key):
    del cfg_key
    anchors = jnp.asarray(_np_generate_anchors(), dtype=scores.dtype)
    sc = scores[:, _A:, :, :]
    B = bbox_deltas.shape[0]
    fh, fw = sc.shape[2], sc.shape[3]
    sx, sy = jnp.meshgrid(jnp.arange(fw) * _FEAT_STRIDE,
                          jnp.arange(fh) * _FEAT_STRIDE)
    shifts = jnp.stack([sx.ravel(), sy.ravel(), sx.ravel(), sy.ravel()],
                       axis=1).astype(scores.dtype)
    K = shifts.shape[0]
    all_anchors = (shifts[:, None, :] + anchors[None, :, :]).reshape(K * _A, 4)
    all_anchors = jnp.broadcast_to(all_anchors[None], (B, K * _A, 4))
    deltas = jnp.transpose(bbox_deltas, (0, 2, 3, 1)).reshape(B, -1, 4)
    scf = jnp.transpose(sc, (0, 2, 3, 1)).reshape(B, -1)
    proposals = _clip_boxes(_bbox_transform_inv(all_anchors, deltas), im_info)

    fpos = _posn_pallas(scf.reshape(B, _ROWS, _SEL_COLS))
    gbox, gsc = _sc_pallas(scf.reshape(-1),
                           fpos.reshape(_B * _ROWS * _NBATCH, 128),
                           proposals.reshape(B * _KA, 4))
    gbox = gbox[:_B * _NPAD].reshape(B, _NPAD, 4)
    gsc = gsc[:_B * _NPAD]
    boxes_in = jnp.transpose(gbox, (0, 2, 1)).reshape(B, 4, _ROWS, _COLS)
    scores_in = gsc.reshape(B, _ROWS, _COLS)

    kept = _nms_pallas(boxes_in, scores_in)
    bcol = jnp.broadcast_to(
        jnp.arange(B, dtype=kept.dtype)[:, None, None], (B, _POST_N, 1))
    return jnp.concatenate([bcol, kept], axis=2)


# speedup vs baseline: 15.6819x; 15.5200x over previous
import functools

import jax
import jax.numpy as jnp
import numpy as np
from jax import lax
from jax.experimental import pallas as pl
from jax.experimental.pallas import tpu as pltpu

_A = 9
_FEAT_STRIDE = 16
_PRE_N = 6000
_POST_N = 300
_THRESH = 0.7
_KA = 36864
_B = 4
_NPAD = 6144
_ROWS = 8
_COLS = _NPAD // _ROWS
_SEL_COLS = _KA // _ROWS
_INT_MIN = np.int32(-2**31)
_DUMP = _B * _NPAD
_GROWS = _B * _NPAD + 128
_TPT = _KA // 8
_NBATCH = _TPT // 128


def _np_generate_anchors():
    base_size = 16
    ratios = np.array([0.5, 1.0, 2.0])
    scales = np.array([8.0, 16.0, 32.0])

    def whctrs(a):
        w = a[2] - a[0] + 1.0
        h = a[3] - a[1] + 1.0
        return w, h, a[0] + 0.5 * (w - 1.0), a[1] + 0.5 * (h - 1.0)

    def mkanchors(ws, hs, xc, yc):
        ws = ws[:, None]
        hs = hs[:, None]
        return np.hstack((xc - 0.5 * (ws - 1.0), yc - 0.5 * (hs - 1.0),
                          xc + 0.5 * (ws - 1.0), yc + 0.5 * (hs - 1.0)))

    base = np.array([1.0, 1.0, base_size, base_size], dtype=np.float64) - 1.0
    w, h, xc, yc = whctrs(base)
    size = w * h
    ws = np.round(np.sqrt(size / ratios))
    hs = np.round(ws * ratios)
    ra = mkanchors(ws, hs, xc, yc)
    outs = []
    for i in range(ra.shape[0]):
        w, h, xc, yc = whctrs(ra[i])
        outs.append(mkanchors(w * scales, h * scales, xc, yc))
    return np.vstack(outs)


def _posn_body(scf_ref, out_ref):
    ri = lax.broadcasted_iota(jnp.int32, (_ROWS, _SEL_COLS), 0)
    ci = lax.broadcasted_iota(jnp.int32, (_ROWS, _SEL_COLS), 1)
    fi = ri * _SEL_COLS + ci

    keys = []
    for b in range(_B):
        bits = lax.bitcast_convert_type(scf_ref[b, :, :], jnp.int32)
        keys.append(jnp.where(bits >= 0, bits, bits ^ jnp.int32(0x7FFFFFFF)))

    pus = [jnp.int32(0)] * _B
    for j in range(31, -1, -1):
        bit = jnp.int32(np.int32(np.uint32(1) << np.uint32(j)))
        qus = [pus[b] | bit for b in range(_B)]
        cnts = [jnp.sum((keys[b] >= (qus[b] ^ _INT_MIN)).astype(jnp.int32))
                for b in range(_B)]
        pus = [jnp.where(cnts[b] >= _PRE_N, qus[b], pus[b])
               for b in range(_B)]
    t_ss = [pus[b] ^ _INT_MIN for b in range(_B)]

    cgts = [jnp.sum((keys[b] > t_ss[b]).astype(jnp.int32))
            for b in range(_B)]
    needs = [_PRE_N - cgts[b] for b in range(_B)]
    ties = [keys[b] == t_ss[b] for b in range(_B)]
    pis = [jnp.int32(0)] * _B
    for j in range(15, -1, -1):
        qis = [pis[b] | jnp.int32(1 << j) for b in range(_B)]
        cnts = [jnp.sum((ties[b] & (fi <= qis[b])).astype(jnp.int32))
                for b in range(_B)]
        pis = [jnp.where(cnts[b] <= needs[b], qis[b], pis[b])
               for b in range(_B)]

    for b in range(_B):
        member = (keys[b] > t_ss[b]) | (ties[b] & (fi <= pis[b]))
        mi = member.astype(jnp.int32)
        c = mi
        sh = 1
        while sh < _SEL_COLS:
            z = jnp.zeros((_ROWS, sh), jnp.int32)
            c = c + jnp.concatenate([z, c[:, :_SEL_COLS - sh]], axis=1)
            sh *= 2
        rowtot = c[:, _SEL_COLS - 1:_SEL_COLS]
        acc = jnp.zeros((1, 1), jnp.int32)
        offs = []
        for r in range(_ROWS):
            offs.append(acc)
            acc = acc + rowtot[r:r + 1, :]
        ro = jnp.concatenate(offs, axis=0)
        excl = c - mi + ro
        out_ref[b, :, :] = jnp.where(member, b * _NPAD + excl,
                                     jnp.int32(_DUMP))


def _posn_pallas(scf):
    return pl.pallas_call(
        _posn_body,
        out_shape=jax.ShapeDtypeStruct((_B, _ROWS, _SEL_COLS), jnp.int32),
    )(scf)


def _nms_body(boxes_ref, scores_ref, out_ref, area_ref):
    ri = lax.broadcasted_iota(jnp.int32, (_ROWS, _COLS), 0)
    ci = lax.broadcasted_iota(jnp.int32, (_ROWS, _COLS), 1)
    fi = ri * _COLS + ci
    neg_inf = jnp.float32(-jnp.inf)
    zero = jnp.float32(0.0)
    big = jnp.int32(2**30)

    s0s = []
    for b in range(_B):
        x1 = boxes_ref[b, 0, :, :]
        y1 = boxes_ref[b, 1, :, :]
        x2 = boxes_ref[b, 2, :, :]
        y2 = boxes_ref[b, 3, :, :]
        area_ref[b, :, :] = (x2 - x1 + 1.0) * (y2 - y1 + 1.0)
        s0s.append(jnp.where(fi < _PRE_N, scores_ref[b, :, :], neg_inf))

    def step(t, ss):
        ms = [jnp.max(ss[b]) for b in range(_B)]
        eqs = [ss[b] == ms[b] for b in range(_B)]
        idxs = [jnp.min(jnp.where(eqs[b], fi, big)) for b in range(_B)]
        sels = [fi == idxs[b] for b in range(_B)]
        x1s = [boxes_ref[b, 0, :, :] for b in range(_B)]
        y1s = [boxes_ref[b, 1, :, :] for b in range(_B)]
        x2s = [boxes_ref[b, 2, :, :] for b in range(_B)]
        y2s = [boxes_ref[b, 3, :, :] for b in range(_B)]
        bx1s = [jnp.sum(jnp.where(sels[b], x1s[b], zero)) for b in range(_B)]
        by1s = [jnp.sum(jnp.where(sels[b], y1s[b], zero)) for b in range(_B)]
        bx2s = [jnp.sum(jnp.where(sels[b], x2s[b], zero)) for b in range(_B)]
        by2s = [jnp.sum(jnp.where(sels[b], y2s[b], zero)) for b in range(_B)]
        out = []
        for b in range(_B):
            x1, y1, x2, y2 = x1s[b], y1s[b], x2s[b], y2s[b]
            areas = area_ref[b, :, :]
            bx1, by1, bx2, by2 = bx1s[b], by1s[b], bx2s[b], by2s[b]
            barea = (bx2 - bx1 + 1.0) * (by2 - by1 + 1.0)
            xx1 = jnp.maximum(x1, bx1)
            yy1 = jnp.maximum(y1, by1)
            xx2 = jnp.minimum(x2, bx2)
            yy2 = jnp.minimum(y2, by2)
            inter = (jnp.maximum(0.0, xx2 - xx1 + 1.0)
                     * jnp.maximum(0.0, yy2 - yy1 + 1.0))
            iou = inter / jnp.maximum(areas + barea - inter, 1e-6)
            out.append(jnp.where(iou <= _THRESH, ss[b], neg_inf))
        for b in range(_B):
            okf = jnp.isfinite(ms[b]).astype(jnp.float32)
            out_ref[b, t, 0] = bx1s[b] * okf
            out_ref[b, t, 1] = by1s[b] * okf
            out_ref[b, t, 2] = bx2s[b] * okf
            out_ref[b, t, 3] = by2s[b] * okf
        return tuple(out)

    lax.fori_loop(0, _POST_N, step, tuple(s0s))


def _nms_pallas(boxes, scores):
    B = boxes.shape[0]
    return pl.pallas_call(
        _nms_body,
        out_specs=pl.BlockSpec(memory_space=pltpu.SMEM),
        out_shape=jax.ShapeDtypeStruct((B, _POST_N, 4), jnp.float32),
        scratch_shapes=[pltpu.VMEM((_B, _ROWS, _COLS), jnp.float32)],
    )(boxes, scores)


def _bbox_transform_inv(boxes, deltas):
    widths = boxes[..., 2] - boxes[..., 0] + 1.0
    heights = boxes[..., 3] - boxes[..., 1] + 1.0
    ctr_x = boxes[..., 0] + 0.5 * widths
    ctr_y = boxes[..., 1] + 0.5 * heights
    dx, dy, dw, dh = (deltas[..., 0], deltas[..., 1],
                      deltas[..., 2], deltas[..., 3])
    pcx = dx * widths + ctr_x
    pcy = dy * heights + ctr_y
    pw = jnp.exp(dw) * widths
    ph = jnp.exp(dh) * heights
    return jnp.stack([pcx - 0.5 * pw, pcy - 0.5 * ph,
                      pcx + 0.5 * pw, pcy + 0.5 * ph], axis=-1)


def _clip_boxes(boxes, im_info):
    hmax = (im_info[:, 0] - 1.0)[:, None]
    wmax = (im_info[:, 1] - 1.0)[:, None]
    x1 = jnp.clip(boxes[..., 0], 0.0, wmax)
    y1 = jnp.clip(boxes[..., 1], 0.0, hmax)
    x2 = jnp.clip(boxes[..., 2], 0.0, wmax)
    y2 = jnp.clip(boxes[..., 3], 0.0, hmax)
    return jnp.stack([x1, y1, x2, y2], axis=-1)


def kernel(scores, bbox_deltas, im_info, cfg_key):
    del cfg_key
    anchors = jnp.asarray(_np_generate_anchors(), dtype=scores.dtype)
    sc = scores[:, _A:, :, :]
    B = bbox_deltas.shape[0]
    fh, fw = sc.shape[2], sc.shape[3]
    sx, sy = jnp.meshgrid(jnp.arange(fw) * _FEAT_STRIDE,
                          jnp.arange(fh) * _FEAT_STRIDE)
    shifts = jnp.stack([sx.ravel(), sy.ravel(), sx.ravel(), sy.ravel()],
                       axis=1).astype(scores.dtype)
    K = shifts.shape[0]
    all_anchors = (shifts[:, None, :] + anchors[None, :, :]).reshape(K * _A, 4)
    all_anchors = jnp.broadcast_to(all_anchors[None], (B, K * _A, 4))
    deltas = jnp.transpose(bbox_deltas, (0, 2, 3, 1)).reshape(B, -1, 4)
    scf = jnp.transpose(sc, (0, 2, 3, 1)).reshape(B, -1)
    proposals = _clip_boxes(_bbox_transform_inv(all_anchors, deltas), im_info)

    fpos = _posn_pallas(scf.reshape(B, _ROWS, _SEL_COLS))
    src_idx = jnp.arange(_B * _KA, dtype=jnp.int32)
    compact = jnp.zeros((_GROWS,), jnp.int32).at[fpos.reshape(-1)].set(src_idx)
    cidx = compact[:_B * _NPAD]
    gbox = proposals.reshape(B * _KA, 4)[cidx].reshape(B, _NPAD, 4)
    gsc = scf.reshape(-1)[cidx]
    boxes_in = jnp.transpose(gbox, (0, 2, 1)).reshape(B, 4, _ROWS, _COLS)
    scores_in = gsc.reshape(B, _ROWS, _COLS)

    kept = _nms_pallas(boxes_in, scores_in)
    bcol = jnp.broadcast_to(
        jnp.arange(B, dtype=kept.dtype)[:, None, None], (B, _POST_N, 1))
    return jnp.concatenate([bcol, kept], axis=2)


# speedup vs baseline: 21.0008x; 1.3392x over previous
import functools

import jax
import jax.numpy as jnp
import numpy as np
from jax.experimental import pallas as pl
from jax.experimental.pallas import tpu as pltpu

_A = 9
_FEAT_STRIDE = 16
_PRE_N = 6000
_POST_N = 300
_THRESH = 0.7
_NPAD = 6144
_B = 4
_ROWS = 8
_COLS = _NPAD // _ROWS


def _np_generate_anchors():
    base_size = 16
    ratios = np.array([0.5, 1.0, 2.0])
    scales = np.array([8.0, 16.0, 32.0])

    def whctrs(a):
        w = a[2] - a[0] + 1.0
        h = a[3] - a[1] + 1.0
        return w, h, a[0] + 0.5 * (w - 1.0), a[1] + 0.5 * (h - 1.0)

    def mkanchors(ws, hs, xc, yc):
        ws = ws[:, None]
        hs = hs[:, None]
        return np.hstack((xc - 0.5 * (ws - 1.0), yc - 0.5 * (hs - 1.0),
                          xc + 0.5 * (ws - 1.0), yc + 0.5 * (hs - 1.0)))

    base = np.array([1.0, 1.0, base_size, base_size], dtype=np.float64) - 1.0
    w, h, xc, yc = whctrs(base)
    size = w * h
    ws = np.round(np.sqrt(size / ratios))
    hs = np.round(ws * ratios)
    ra = mkanchors(ws, hs, xc, yc)
    outs = []
    for i in range(ra.shape[0]):
        w, h, xc, yc = whctrs(ra[i])
        outs.append(mkanchors(w * scales, h * scales, xc, yc))
    return np.vstack(outs)


def _nms_body(boxes_ref, scores_ref, out_ref, area_ref):
    ri = jax.lax.broadcasted_iota(jnp.int32, (_ROWS, _COLS), 0)
    ci = jax.lax.broadcasted_iota(jnp.int32, (_ROWS, _COLS), 1)
    fi = ri * _COLS + ci
    neg_inf = jnp.float32(-jnp.inf)
    zero = jnp.float32(0.0)
    big = jnp.int32(2**30)

    s0s = []
    for b in range(_B):
        x1 = boxes_ref[b, 0, :, :]
        y1 = boxes_ref[b, 1, :, :]
        x2 = boxes_ref[b, 2, :, :]
        y2 = boxes_ref[b, 3, :, :]
        area_ref[b, :, :] = (x2 - x1 + 1.0) * (y2 - y1 + 1.0)
        s0s.append(scores_ref[b, :, :])

    def step(t, ss):
        ms = [jnp.max(ss[b]) for b in range(_B)]
        eqs = [ss[b] == ms[b] for b in range(_B)]
        idxs = [jnp.min(jnp.where(eqs[b], fi, big)) for b in range(_B)]
        sels = [fi == idxs[b] for b in range(_B)]
        x1s = [boxes_ref[b, 0, :, :] for b in range(_B)]
        y1s = [boxes_ref[b, 1, :, :] for b in range(_B)]
        x2s = [boxes_ref[b, 2, :, :] for b in range(_B)]
        y2s = [boxes_ref[b, 3, :, :] for b in range(_B)]
        bx1s = [jnp.sum(jnp.where(sels[b], x1s[b], zero)) for b in range(_B)]
        by1s = [jnp.sum(jnp.where(sels[b], y1s[b], zero)) for b in range(_B)]
        bx2s = [jnp.sum(jnp.where(sels[b], x2s[b], zero)) for b in range(_B)]
        by2s = [jnp.sum(jnp.where(sels[b], y2s[b], zero)) for b in range(_B)]
        out = []
        for b in range(_B):
            x1, y1, x2, y2 = x1s[b], y1s[b], x2s[b], y2s[b]
            areas = area_ref[b, :, :]
            bx1, by1, bx2, by2 = bx1s[b], by1s[b], bx2s[b], by2s[b]
            barea = (bx2 - bx1 + 1.0) * (by2 - by1 + 1.0)
            xx1 = jnp.maximum(x1, bx1)
            yy1 = jnp.maximum(y1, by1)
            xx2 = jnp.minimum(x2, bx2)
            yy2 = jnp.minimum(y2, by2)
            inter = (jnp.maximum(0.0, xx2 - xx1 + 1.0)
                     * jnp.maximum(0.0, yy2 - yy1 + 1.0))
            iou = inter / jnp.maximum(areas + barea - inter, 1e-6)
            out.append(jnp.where(iou <= _THRESH, ss[b], neg_inf))
        for b in range(_B):
            okf = jnp.isfinite(ms[b]).astype(jnp.float32)
            out_ref[b, t, 0] = bx1s[b] * okf
            out_ref[b, t, 1] = by1s[b] * okf
            out_ref[b, t, 2] = bx2s[b] * okf
            out_ref[b, t, 3] = by2s[b] * okf
        return tuple(out)

    jax.lax.fori_loop(0, _POST_N, step, tuple(s0s))


@functools.partial(jax.jit, static_argnums=())
def _nms_pallas(boxes, scores):
    B = boxes.shape[0]
    return pl.pallas_call(
        _nms_body,
        out_specs=pl.BlockSpec(memory_space=pltpu.SMEM),
        out_shape=jax.ShapeDtypeStruct((B, _POST_N, 4), jnp.float32),
        scratch_shapes=[pltpu.VMEM((_B, _ROWS, _COLS), jnp.float32)],
    )(boxes, scores)


def _bbox_transform_inv(boxes, deltas):
    widths = boxes[..., 2] - boxes[..., 0] + 1.0
    heights = boxes[..., 3] - boxes[..., 1] + 1.0
    ctr_x = boxes[..., 0] + 0.5 * widths
    ctr_y = boxes[..., 1] + 0.5 * heights
    dx, dy, dw, dh = (deltas[..., 0], deltas[..., 1],
                      deltas[..., 2], deltas[..., 3])
    pcx = dx * widths + ctr_x
    pcy = dy * heights + ctr_y
    pw = jnp.exp(dw) * widths
    ph = jnp.exp(dh) * heights
    return jnp.stack([pcx - 0.5 * pw, pcy - 0.5 * ph,
                      pcx + 0.5 * pw, pcy + 0.5 * ph], axis=-1)


def _clip_boxes(boxes, im_info):
    hmax = (im_info[:, 0] - 1.0)[:, None]
    wmax = (im_info[:, 1] - 1.0)[:, None]
    x1 = jnp.clip(boxes[..., 0], 0.0, wmax)
    y1 = jnp.clip(boxes[..., 1], 0.0, hmax)
    x2 = jnp.clip(boxes[..., 2], 0.0, wmax)
    y2 = jnp.clip(boxes[..., 3], 0.0, hmax)
    return jnp.stack([x1, y1, x2, y2], axis=-1)


def kernel(scores, bbox_deltas, im_info, cfg_key):
    del cfg_key
    anchors = jnp.asarray(_np_generate_anchors(), dtype=scores.dtype)
    sc = scores[:, _A:, :, :]
    B = bbox_deltas.shape[0]
    fh, fw = sc.shape[2], sc.shape[3]
    sx, sy = jnp.meshgrid(jnp.arange(fw) * _FEAT_STRIDE,
                          jnp.arange(fh) * _FEAT_STRIDE)
    shifts = jnp.stack([sx.ravel(), sy.ravel(), sx.ravel(), sy.ravel()],
                       axis=1).astype(scores.dtype)
    K = shifts.shape[0]
    all_anchors = (shifts[:, None, :] + anchors[None, :, :]).reshape(K * _A, 4)
    all_anchors = jnp.broadcast_to(all_anchors[None], (B, K * _A, 4))
    deltas = jnp.transpose(bbox_deltas, (0, 2, 3, 1)).reshape(B, -1, 4)
    scf = jnp.transpose(sc, (0, 2, 3, 1)).reshape(B, -1)
    proposals = _clip_boxes(_bbox_transform_inv(all_anchors, deltas), im_info)
    top_scores, order = jax.lax.top_k(scf, _PRE_N)
    props = jnp.take_along_axis(proposals, order[:, :, None], axis=1)

    pad_n = _NPAD - _PRE_N
    props_p = jnp.concatenate(
        [props, jnp.zeros((B, pad_n, 4), props.dtype)], axis=1)
    scores_p = jnp.concatenate(
        [top_scores, jnp.full((B, pad_n), -jnp.inf, top_scores.dtype)], axis=1)

    boxes_in = jnp.transpose(props_p, (0, 2, 1)).reshape(B, 4, _ROWS, _COLS)
    scores_in = scores_p.reshape(B, _ROWS, _COLS)

    kept = _nms_pallas(boxes_in, scores_in)
    bcol = jnp.broadcast_to(
        jnp.arange(B, dtype=kept.dtype)[:, None, None], (B, _POST_N, 1))
    return jnp.concatenate([bcol, kept], axis=2)
